# Initial kernel scaffold; baseline (speedup 1.0000x reference)
import jax
import jax.numpy as jnp
from jax.experimental import pallas as pl


def kernel(x, edge_index, Wl1, Wr1, b1, Wl2, Wr2, b2):
    raise NotImplementedError("write your pallas kernel here")


# speedup vs baseline: 3.8003x; 3.8003x over previous
import jax
import jax.numpy as jnp
from jax import lax
from jax.experimental import pallas as pl
from jax.experimental.pallas import tpu as pltpu
from jax.experimental.pallas import tpu_sc as plsc

N_NODES = 10000
N_PAD = 10112
ROWS_PER_TILE = N_PAD // 16
N_EDGES = 320000
CHUNK = 128
EDGE_ROWS = 2560
ROWS_PER_WORKER = EDGE_ROWS // 32
IDX_STAGE = 16

NC, NS = 2, 16
NW = NC * NS


def _zero_fill(buf, n_rows, cols):
    zeros16 = jnp.zeros((16,), jnp.float32)

    @pl.loop(0, n_rows * (cols // 16))
    def _(i):
        r = i // (cols // 16)
        c = (i % (cols // 16)) * 16
        buf[r, pl.ds(c, 16)] = zeros16


def _make_segsum(d_feat, with_cnt):
    mesh = plsc.VectorSubcoreMesh(core_axis_name="c", subcore_axis_name="s",
                                  num_cores=NC, num_subcores=NS)
    out_type = [jax.ShapeDtypeStruct((NC, N_PAD, d_feat), jnp.float32)]
    if with_cnt:
        out_type.append(jax.ShapeDtypeStruct((NW * N_PAD,), jnp.float32))
    n_stage = ROWS_PER_WORKER // IDX_STAGE
    scratch = [
        pltpu.VMEM_SHARED((N_PAD, d_feat), jnp.float32),
        pltpu.VMEM((IDX_STAGE, CHUNK), jnp.int32),
        pltpu.VMEM((IDX_STAGE, CHUNK), jnp.int32),
        pltpu.VMEM((CHUNK, d_feat), jnp.float32),
        pltpu.SemaphoreType.DMA,
    ]
    if with_cnt:
        scratch.append(pltpu.VMEM((N_PAD,), jnp.float32))

    def body(feat, src2d, dst2d, *rest):
        if with_cnt:
            (out, cout, acc, srcbuf, dstbuf, rows, gsem, cnt_local) = rest
        else:
            (out, acc, srcbuf, dstbuf, rows, gsem) = rest
            cout = cnt_local = None
        c = lax.axis_index("c")
        s = lax.axis_index("s")
        wid = s * NC + c

        _zero_fill(rows, CHUNK, d_feat)
        if with_cnt:
            zeros16 = jnp.zeros((16,), jnp.float32)

            @pl.loop(0, N_PAD // 16)
            def _(i):
                cnt_local[pl.ds(i * 16, 16)] = zeros16

        base = s * ROWS_PER_TILE
        full, rem = ROWS_PER_TILE // CHUNK, ROWS_PER_TILE % CHUNK
        for k in range(full):
            pltpu.sync_copy(rows, acc.at[pl.ds(base + k * CHUNK, CHUNK)])
        if rem:
            pltpu.sync_copy(rows.at[pl.ds(0, rem)],
                            acc.at[pl.ds(base + full * CHUNK, rem)])
        plsc.subcore_barrier()

        ebase = wid * ROWS_PER_WORKER
        ones16 = jnp.ones((16,), jnp.float32)

        @pl.loop(0, n_stage)
        def _(st):
            pltpu.sync_copy(src2d.at[pl.ds(ebase + st * IDX_STAGE, IDX_STAGE)],
                            srcbuf)
            pltpu.sync_copy(dst2d.at[pl.ds(ebase + st * IDX_STAGE, IDX_STAGE)],
                            dstbuf)

            @pl.loop(0, IDX_STAGE)
            def _(j):
                pltpu.async_copy(feat.at[srcbuf.at[j]], rows, gsem).wait()
                pltpu.sync_copy(rows, acc.at[dstbuf.at[j]], add=True)
                if with_cnt:
                    for k in range(CHUNK // 16):
                        idx = dstbuf[j, pl.ds(k * 16, 16)]
                        plsc.addupdate_scatter(cnt_local, [idx], ones16)

        plsc.subcore_barrier()

        pltpu.sync_copy(acc.at[pl.ds(base, ROWS_PER_TILE)],
                        out.at[c, pl.ds(base, ROWS_PER_TILE)])
        if with_cnt:
            pltpu.sync_copy(cnt_local, cout.at[pl.ds(wid * N_PAD, N_PAD)])

    return pl.kernel(body, out_type=out_type, mesh=mesh, scratch_types=scratch,
                     compiler_params=pltpu.CompilerParams(needs_layout_passes=False))


_segsum_l1 = _make_segsum(128, with_cnt=True)
_segsum_l2 = _make_segsum(128, with_cnt=False)

_ROW_BLK = 1000


def _tc1_body(s1_ref, cnt_ref, x_ref, wl1_ref, wr1_ref, b1_ref, wl2_ref,
              wr2_ref, b2_ref, p_ref, q_ref):
    tot = jnp.maximum(jnp.sum(cnt_ref[...], axis=1), 1.0)
    agg = (s1_ref[0] + s1_ref[1]) / tot[:, None]
    h = agg @ wl1_ref[...] + x_ref[...] @ wr1_ref[...] + b1_ref[...]
    h = jnp.maximum(h, 0.0)
    p_ref[...] = jnp.concatenate(
        [h @ wl2_ref[...], jnp.zeros((h.shape[0], 64), jnp.float32)], axis=1)
    q_ref[...] = h @ wr2_ref[...] + b2_ref[...]


def _tc2_body(s2_ref, cnt_ref, q_ref, o_ref):
    tot = jnp.maximum(jnp.sum(cnt_ref[...], axis=1), 1.0)
    z = (s2_ref[0] + s2_ref[1]) / tot[:, None] + q_ref[...]
    m = jnp.max(z, axis=1, keepdims=True)
    e = jnp.exp(z - m)
    o_ref[...] = (z - m) - jnp.log(jnp.sum(e, axis=1, keepdims=True))


def _tc1(s1, cnt_t, x, Wl1, Wr1, b1, Wl2, Wr2, b2):
    n_blk = N_NODES // _ROW_BLK
    blk = lambda shape, imap: pl.BlockSpec(shape, imap)
    return pl.pallas_call(
        _tc1_body,
        grid=(n_blk,),
        in_specs=[
            blk((2, _ROW_BLK, 128), lambda i: (0, i, 0)),
            blk((_ROW_BLK, NW), lambda i: (i, 0)),
            blk((_ROW_BLK, 128), lambda i: (i, 0)),
            blk((128, 256), lambda i: (0, 0)),
            blk((128, 256), lambda i: (0, 0)),
            blk((1, 256), lambda i: (0, 0)),
            blk((256, 64), lambda i: (0, 0)),
            blk((256, 64), lambda i: (0, 0)),
            blk((1, 64), lambda i: (0, 0)),
        ],
        out_specs=[
            blk((_ROW_BLK, 128), lambda i: (i, 0)),
            blk((_ROW_BLK, 64), lambda i: (i, 0)),
        ],
        out_shape=[
            jax.ShapeDtypeStruct((N_NODES, 128), jnp.float32),
            jax.ShapeDtypeStruct((N_NODES, 64), jnp.float32),
        ],
    )(s1, cnt_t, x, Wl1, Wr1, b1.reshape(1, 256), Wl2, Wr2, b2.reshape(1, 64))


def _tc2(s2, cnt_t, q):
    n_blk = N_NODES // _ROW_BLK
    blk = lambda shape, imap: pl.BlockSpec(shape, imap)
    return pl.pallas_call(
        _tc2_body,
        grid=(n_blk,),
        in_specs=[
            blk((2, _ROW_BLK, 64), lambda i: (0, i, 0)),
            blk((_ROW_BLK, NW), lambda i: (i, 0)),
            blk((_ROW_BLK, 64), lambda i: (i, 0)),
        ],
        out_specs=blk((_ROW_BLK, 64), lambda i: (i, 0)),
        out_shape=jax.ShapeDtypeStruct((N_NODES, 64), jnp.float32),
    )(s2, cnt_t, q)


def kernel(x, edge_index, Wl1, Wr1, b1, Wl2, Wr2, b2):
    src = edge_index[0].astype(jnp.int32)
    dst = edge_index[1].astype(jnp.int32)
    pad = EDGE_ROWS * CHUNK - N_EDGES
    src2d = jnp.concatenate(
        [src, jnp.zeros((pad,), jnp.int32)]).reshape(EDGE_ROWS, CHUNK)
    dst2d = jnp.concatenate(
        [dst, jnp.full((pad,), N_NODES, jnp.int32)]).reshape(EDGE_ROWS, CHUNK)

    s1p, cntp = _segsum_l1(x, src2d, dst2d)
    cnt_t = jnp.transpose(cntp.reshape(NW, N_PAD)[:, :N_NODES])
    p, q = _tc1(s1p[:, :N_NODES], cnt_t, x, Wl1, Wr1, b1, Wl2, Wr2, b2)
    (s2p,) = _segsum_l2(p, src2d, dst2d)
    return _tc2(s2p[:, :N_NODES, :64], cnt_t, q)


# speedup vs baseline: 4.4596x; 1.1735x over previous
import jax
import jax.numpy as jnp
from jax import lax
from jax.experimental import pallas as pl
from jax.experimental.pallas import tpu as pltpu
from jax.experimental.pallas import tpu_sc as plsc

N_NODES = 10000
N_PAD = 10112
ROWS_PER_TILE = N_PAD // 16
N_EDGES = 320000
CHUNK = 64
EDGE_ROWS = 5120
ROWS_PER_WORKER = EDGE_ROWS // 32

NC, NS = 2, 16
NW = NC * NS


def _zero_fill(buf, n_rows, cols):
    zeros16 = jnp.zeros((16,), jnp.float32)

    @pl.loop(0, n_rows * (cols // 16))
    def _(i):
        r = i // (cols // 16)
        c = (i % (cols // 16)) * 16
        buf[r, pl.ds(c, 16)] = zeros16


def _make_segsum(d_feat, with_cnt):
    mesh = plsc.VectorSubcoreMesh(core_axis_name="c", subcore_axis_name="s",
                                  num_cores=NC, num_subcores=NS)
    out_type = [jax.ShapeDtypeStruct((NC, N_PAD, d_feat), jnp.float32)]
    if with_cnt:
        out_type.append(jax.ShapeDtypeStruct((NW * N_PAD,), jnp.float32))
    half = ROWS_PER_WORKER // 2
    scratch = [
        pltpu.VMEM_SHARED((N_PAD, d_feat), jnp.float32),
        pltpu.VMEM((half, CHUNK), jnp.int32),
        pltpu.VMEM((half, CHUNK), jnp.int32),
        pltpu.VMEM((CHUNK, d_feat), jnp.float32),
        pltpu.VMEM((CHUNK, d_feat), jnp.float32),
        pltpu.SemaphoreType.DMA,
        pltpu.SemaphoreType.DMA,
    ]
    if with_cnt:
        scratch.append(pltpu.VMEM((N_PAD,), jnp.float32))

    def body(feat, src2d, dst2d, *rest):
        if with_cnt:
            (out, cout, acc, srcbuf, dstbuf, rows0, rows1, gsem0, gsem1,
             cnt_local) = rest
        else:
            (out, acc, srcbuf, dstbuf, rows0, rows1, gsem0, gsem1) = rest
            cout = cnt_local = None
        c = lax.axis_index("c")
        s = lax.axis_index("s")
        wid = s * NC + c

        _zero_fill(rows0, CHUNK, d_feat)
        if with_cnt:
            zeros16 = jnp.zeros((16,), jnp.float32)

            @pl.loop(0, N_PAD // 16)
            def _(i):
                cnt_local[pl.ds(i * 16, 16)] = zeros16

        base = s * ROWS_PER_TILE
        full, rem = ROWS_PER_TILE // CHUNK, ROWS_PER_TILE % CHUNK
        for k in range(full):
            pltpu.sync_copy(rows0, acc.at[pl.ds(base + k * CHUNK, CHUNK)])
        if rem:
            pltpu.sync_copy(rows0.at[pl.ds(0, rem)],
                            acc.at[pl.ds(base + full * CHUNK, rem)])
        plsc.subcore_barrier()

        ebase = wid * ROWS_PER_WORKER
        ones16 = jnp.ones((16,), jnp.float32)

        def scatter(rows, j):
            pltpu.sync_copy(rows, acc.at[dstbuf.at[j]], add=True)
            if with_cnt:
                for k in range(CHUNK // 16):
                    idx = dstbuf[j, pl.ds(k * 16, 16)]
                    plsc.addupdate_scatter(cnt_local, [idx], ones16)

        for h in range(2):
            pltpu.sync_copy(src2d.at[pl.ds(ebase + h * half, half)], srcbuf)
            pltpu.sync_copy(dst2d.at[pl.ds(ebase + h * half, half)], dstbuf)
            pltpu.async_copy(feat.at[srcbuf.at[0]], rows0, gsem0)

            @pl.loop(0, half // 2 - 1)
            def _(i):
                j = i * 2
                pltpu.async_copy(feat.at[srcbuf.at[j + 1]], rows1, gsem1)
                pltpu.make_async_copy(feat.at[srcbuf.at[j]], rows0, gsem0).wait()
                scatter(rows0, j)
                pltpu.async_copy(feat.at[srcbuf.at[j + 2]], rows0, gsem0)
                pltpu.make_async_copy(feat.at[srcbuf.at[j + 1]], rows1,
                                      gsem1).wait()
                scatter(rows1, j + 1)

            jl = half - 2
            pltpu.async_copy(feat.at[srcbuf.at[jl + 1]], rows1, gsem1)
            pltpu.make_async_copy(feat.at[srcbuf.at[jl]], rows0, gsem0).wait()
            scatter(rows0, jl)
            pltpu.make_async_copy(feat.at[srcbuf.at[jl + 1]], rows1, gsem1).wait()
            scatter(rows1, jl + 1)

        plsc.subcore_barrier()

        pltpu.sync_copy(acc.at[pl.ds(base, ROWS_PER_TILE)],
                        out.at[c, pl.ds(base, ROWS_PER_TILE)])
        if with_cnt:
            pltpu.sync_copy(cnt_local, cout.at[pl.ds(wid * N_PAD, N_PAD)])

    return pl.kernel(body, out_type=out_type, mesh=mesh, scratch_types=scratch,
                     compiler_params=pltpu.CompilerParams(needs_layout_passes=False))


_segsum_l1 = _make_segsum(128, with_cnt=True)
_segsum_l2 = _make_segsum(128, with_cnt=False)

_ROW_BLK = 1000


def _tc1_body(s1_ref, cnt_ref, x_ref, wl1_ref, wr1_ref, b1_ref, wl2_ref,
              wr2_ref, b2_ref, p_ref, q_ref):
    tot = jnp.maximum(jnp.sum(cnt_ref[...], axis=1), 1.0)
    agg = (s1_ref[0] + s1_ref[1]) / tot[:, None]
    h = agg @ wl1_ref[...] + x_ref[...] @ wr1_ref[...] + b1_ref[...]
    h = jnp.maximum(h, 0.0)
    p_ref[...] = jnp.concatenate(
        [h @ wl2_ref[...], jnp.zeros((h.shape[0], 64), jnp.float32)], axis=1)
    q_ref[...] = h @ wr2_ref[...] + b2_ref[...]


def _tc2_body(s2_ref, cnt_ref, q_ref, o_ref):
    tot = jnp.maximum(jnp.sum(cnt_ref[...], axis=1), 1.0)
    z = (s2_ref[0] + s2_ref[1]) / tot[:, None] + q_ref[...]
    m = jnp.max(z, axis=1, keepdims=True)
    e = jnp.exp(z - m)
    o_ref[...] = (z - m) - jnp.log(jnp.sum(e, axis=1, keepdims=True))


def _tc1(s1, cnt_t, x, Wl1, Wr1, b1, Wl2, Wr2, b2):
    n_blk = N_NODES // _ROW_BLK
    blk = lambda shape, imap: pl.BlockSpec(shape, imap)
    return pl.pallas_call(
        _tc1_body,
        grid=(n_blk,),
        in_specs=[
            blk((2, _ROW_BLK, 128), lambda i: (0, i, 0)),
            blk((_ROW_BLK, NW), lambda i: (i, 0)),
            blk((_ROW_BLK, 128), lambda i: (i, 0)),
            blk((128, 256), lambda i: (0, 0)),
            blk((128, 256), lambda i: (0, 0)),
            blk((1, 256), lambda i: (0, 0)),
            blk((256, 64), lambda i: (0, 0)),
            blk((256, 64), lambda i: (0, 0)),
            blk((1, 64), lambda i: (0, 0)),
        ],
        out_specs=[
            blk((_ROW_BLK, 128), lambda i: (i, 0)),
            blk((_ROW_BLK, 64), lambda i: (i, 0)),
        ],
        out_shape=[
            jax.ShapeDtypeStruct((N_NODES, 128), jnp.float32),
            jax.ShapeDtypeStruct((N_NODES, 64), jnp.float32),
        ],
    )(s1, cnt_t, x, Wl1, Wr1, b1.reshape(1, 256), Wl2, Wr2, b2.reshape(1, 64))


def _tc2(s2, cnt_t, q):
    n_blk = N_NODES // _ROW_BLK
    blk = lambda shape, imap: pl.BlockSpec(shape, imap)
    return pl.pallas_call(
        _tc2_body,
        grid=(n_blk,),
        in_specs=[
            blk((2, _ROW_BLK, 64), lambda i: (0, i, 0)),
            blk((_ROW_BLK, NW), lambda i: (i, 0)),
            blk((_ROW_BLK, 64), lambda i: (i, 0)),
        ],
        out_specs=blk((_ROW_BLK, 64), lambda i: (i, 0)),
        out_shape=jax.ShapeDtypeStruct((N_NODES, 64), jnp.float32),
    )(s2, cnt_t, q)


def kernel(x, edge_index, Wl1, Wr1, b1, Wl2, Wr2, b2):
    src = edge_index[0].astype(jnp.int32)
    dst = edge_index[1].astype(jnp.int32)
    pad = EDGE_ROWS * CHUNK - N_EDGES
    src2d = jnp.concatenate(
        [src, jnp.zeros((pad,), jnp.int32)]).reshape(EDGE_ROWS, CHUNK)
    dst2d = jnp.concatenate(
        [dst, jnp.full((pad,), N_NODES, jnp.int32)]).reshape(EDGE_ROWS, CHUNK)

    s1p, cntp = _segsum_l1(x, src2d, dst2d)
    cnt_t = jnp.transpose(cntp.reshape(NW, N_PAD)[:, :N_NODES])
    p, q = _tc1(s1p[:, :N_NODES], cnt_t, x, Wl1, Wr1, b1, Wl2, Wr2, b2)
    (s2p,) = _segsum_l2(p, src2d, dst2d)
    return _tc2(s2p[:, :N_NODES, :64], cnt_t, q)
